# baseline (speedup 1.0000x reference)
import functools

import jax
import jax.numpy as jnp
from jax import lax
from jax.experimental import pallas as pl
from jax.experimental.pallas import tpu as pltpu
from jax.experimental.pallas import tpu_sc as plsc

N_EMB = 8192
DIM = 256
N_TOK = 16384
HALF = N_TOK // 2
BLK = 512
N_BLK_HALF = HALF // BLK

NC, NS = 2, 16
NW = NC * NS
B_PER_W = HALF // NW
CHUNK = 128

WINDOWS = [(0, 2816), (2816, 5632), (5632, 8192)]


def _dist_argmin_kernel(x_ref, e_ref, idx_ref, loss_ref, enorm_ref):
    i = pl.program_id(0)
    xb = x_ref[...]
    e = e_ref[...]

    @pl.when(i == 0)
    def _():
        enorm_ref[...] = jnp.sum(e * e, axis=0, keepdims=True)

    sim = jnp.dot(xb, e, preferred_element_type=jnp.float32)
    xnorm = jnp.sum(xb * xb, axis=1, keepdims=True)
    enorm = enorm_ref[...]
    dist = (xnorm + enorm) - 2.0 * sim
    acc_v = jnp.full((BLK,), jnp.inf, dtype=jnp.float32)
    acc_i = jnp.zeros((BLK,), dtype=jnp.int32)
    acc_d = jnp.zeros((BLK,), dtype=jnp.float32)
    for lo, hi in WINDOWS:
        sl = dist[:, lo:hi]
        wm = jnp.min(sl, axis=1)
        iota = lax.broadcasted_iota(jnp.int32, sl.shape, 1) + lo
        wi = jnp.min(jnp.where(sl == wm[:, None], iota, N_EMB), axis=1)
        win = wm < acc_v
        acc_i = jnp.where(win, wi, acc_i)
        acc_d = jnp.where(win, wm, acc_d)
        acc_v = jnp.where(win, wm.astype(jnp.bfloat16).astype(jnp.float32),
                          acc_v)
    idx_ref[...] = acc_i[:, None]
    part = jnp.sum(acc_d).reshape(1, 1)
    loss_ref[...] = jnp.where(i == 0, part, loss_ref[...] + part)


def _indices_and_loss(flat, embeddings):
    return pl.pallas_call(
        _dist_argmin_kernel,
        grid=(N_BLK_HALF,),
        in_specs=[
            pl.BlockSpec((BLK, DIM), lambda i: (i, 0)),
            pl.BlockSpec((DIM, N_EMB), lambda i: (0, 0)),
        ],
        out_specs=[
            pl.BlockSpec((BLK, 1), lambda i: (i, 0)),
            pl.BlockSpec((1, 1), lambda i: (0, 0)),
        ],
        out_shape=[
            jax.ShapeDtypeStruct((HALF, 1), jnp.int32),
            jax.ShapeDtypeStruct((1, 1), jnp.float32),
        ],
        scratch_shapes=[pltpu.VMEM((1, N_EMB), jnp.float32)],
    )(flat, embeddings)


@functools.cache
def _make_sc_gather():
    @functools.partial(
        pl.kernel,
        mesh=plsc.VectorSubcoreMesh(core_axis_name="c", subcore_axis_name="s"),
        out_type=jax.ShapeDtypeStruct((HALF, DIM), jnp.float32),
        scratch_types=[
            pltpu.VMEM((B_PER_W,), jnp.int32),
            pltpu.VMEM((CHUNK, DIM), jnp.float32),
            pltpu.SemaphoreType.DMA,
        ],
    )
    def _sc_gather(table_hbm, idx_hbm, out_hbm, idx_v, rows_v, sem):
        wid = lax.axis_index("s") * NC + lax.axis_index("c")
        base = wid * B_PER_W
        pltpu.sync_copy(idx_hbm.at[pl.ds(base, B_PER_W)], idx_v)

        @pl.loop(0, B_PER_W, step=CHUNK)
        def _(c):
            pltpu.async_copy(table_hbm.at[idx_v.at[pl.ds(c, CHUNK)]],
                             rows_v, sem).wait()
            pltpu.sync_copy(rows_v, out_hbm.at[pl.ds(base + c, CHUNK)])

    return _sc_gather


def kernel(x, embeddings):
    flat = x.reshape(N_TOK, DIM)
    table = embeddings.T
    gather = _make_sc_gather()
    halves = []
    for h in range(2):
        idx2, loss_acc = _indices_and_loss(
            flat[h * HALF:(h + 1) * HALF], embeddings)
        idx = idx2.reshape(HALF)
        halves.append((idx, loss_acc, gather(table, idx)))
    idx = jnp.concatenate([halves[0][0], halves[1][0]])
    quantized = jnp.concatenate([halves[0][2], halves[1][2]]).reshape(x.shape)
    loss = (1.25 / (N_TOK * DIM)) * (halves[0][1][0, 0] + halves[1][1][0, 0])
    return quantized, loss, idx


# speedup vs baseline: 1.1057x; 1.1057x over previous
import functools

import jax
import jax.numpy as jnp
from jax import lax
from jax.experimental import pallas as pl
from jax.experimental.pallas import tpu as pltpu
from jax.experimental.pallas import tpu_sc as plsc

N_EMB = 8192
DIM = 256
N_TOK = 16384
HALF = N_TOK // 2
BLK = 512
N_BLK_HALF = HALF // BLK

NC, NS = 2, 16
NW = NC * NS
B_PER_W = HALF // NW
CHUNK = 128

WINDOWS = [(0, 2816), (2816, 5632), (5632, 8192)]


def _dist_argmin_kernel(x_ref, e_ref, iota_ref, idx_ref, loss_ref, enorm_ref):
    i = pl.program_id(0)
    xb = x_ref[...]
    e = e_ref[...]

    @pl.when(i == 0)
    def _():
        enorm_ref[...] = jnp.sum(e * e, axis=0, keepdims=True)

    sim = jnp.dot(xb, e, preferred_element_type=jnp.float32)
    xnorm = jnp.sum(xb * xb, axis=1, keepdims=True)
    enorm = enorm_ref[...]
    dist = (xnorm + enorm) - 2.0 * sim
    acc_v = jnp.full((BLK,), jnp.inf, dtype=jnp.float32)
    acc_i = jnp.zeros((BLK,), dtype=jnp.float32)
    acc_d = jnp.zeros((BLK,), dtype=jnp.float32)
    for lo, hi in WINDOWS:
        sl = dist[:, lo:hi]
        wm = jnp.min(sl, axis=1)
        iota = iota_ref[:, lo:hi]
        wi = jnp.min(jnp.where(sl == wm[:, None], iota, float(N_EMB)), axis=1)
        win = wm < acc_v
        acc_i = jnp.where(win, wi, acc_i)
        acc_d = jnp.where(win, wm, acc_d)
        acc_v = jnp.where(win, wm.astype(jnp.bfloat16).astype(jnp.float32),
                          acc_v)
    idx_ref[...] = acc_i.astype(jnp.int32)[:, None]
    part = jnp.sum(acc_d).reshape(1, 1)
    loss_ref[...] = jnp.where(i == 0, part, loss_ref[...] + part)


def _indices_and_loss(flat, embeddings, iota_row):
    return pl.pallas_call(
        _dist_argmin_kernel,
        grid=(N_BLK_HALF,),
        in_specs=[
            pl.BlockSpec((BLK, DIM), lambda i: (i, 0)),
            pl.BlockSpec((DIM, N_EMB), lambda i: (0, 0)),
            pl.BlockSpec((1, N_EMB), lambda i: (0, 0)),
        ],
        out_specs=[
            pl.BlockSpec((BLK, 1), lambda i: (i, 0)),
            pl.BlockSpec((1, 1), lambda i: (0, 0)),
        ],
        out_shape=[
            jax.ShapeDtypeStruct((HALF, 1), jnp.int32),
            jax.ShapeDtypeStruct((1, 1), jnp.float32),
        ],
        scratch_shapes=[pltpu.VMEM((1, N_EMB), jnp.float32)],
    )(flat, embeddings, iota_row)


@functools.cache
def _make_sc_gather():
    @functools.partial(
        pl.kernel,
        mesh=plsc.VectorSubcoreMesh(core_axis_name="c", subcore_axis_name="s"),
        out_type=jax.ShapeDtypeStruct((HALF, DIM), jnp.float32),
        scratch_types=[
            pltpu.VMEM((B_PER_W,), jnp.int32),
            pltpu.VMEM((CHUNK, DIM), jnp.float32),
            pltpu.SemaphoreType.DMA,
        ],
    )
    def _sc_gather(table_hbm, idx_hbm, out_hbm, idx_v, rows_v, sem):
        wid = lax.axis_index("s") * NC + lax.axis_index("c")
        base = wid * B_PER_W
        pltpu.sync_copy(idx_hbm.at[pl.ds(base, B_PER_W)], idx_v)

        @pl.loop(0, B_PER_W, step=CHUNK)
        def _(c):
            pltpu.async_copy(table_hbm.at[idx_v.at[pl.ds(c, CHUNK)]],
                             rows_v, sem).wait()
            pltpu.sync_copy(rows_v, out_hbm.at[pl.ds(base + c, CHUNK)])

    return _sc_gather


def kernel(x, embeddings):
    flat = x.reshape(N_TOK, DIM)
    table = embeddings.T
    iota_row = lax.broadcasted_iota(jnp.float32, (1, N_EMB), 1)
    gather = _make_sc_gather()
    halves = []
    for h in range(2):
        idx2, loss_acc = _indices_and_loss(
            flat[h * HALF:(h + 1) * HALF], embeddings, iota_row)
        idx = idx2.reshape(HALF)
        halves.append((idx, loss_acc, gather(table, idx)))
    idx = jnp.concatenate([halves[0][0], halves[1][0]])
    quantized = jnp.concatenate([halves[0][2], halves[1][2]]).reshape(x.shape)
    loss = (1.25 / (N_TOK * DIM)) * (halves[0][1][0, 0] + halves[1][1][0, 0])
    return quantized, loss, idx
